# Initial kernel scaffold; baseline (speedup 1.0000x reference)
import jax
import jax.numpy as jnp
from jax.experimental import pallas as pl


def kernel(x_tx, x_acc, x_mer, e_sent, e_recv, e_at, e_rev_sent, e_rev_recv, e_rev_at, params):
    raise NotImplementedError("write your pallas kernel here")


# speedup vs baseline: 1.2445x; 1.2445x over previous
import functools

import jax
import jax.numpy as jnp
from jax import lax
from jax.experimental import pallas as pl
from jax.experimental.pallas import tpu as pltpu
from jax.experimental.pallas import tpu_sc as plsc

H = 4
C = 32
HID = 32
HC = H * C
N_TX, N_ACC, N_MER = 100000, 50000, 10000
E = 100000
NNODE = {"tx": N_TX, "acc": N_ACC, "mer": N_MER}
RELS = [("sent", "tx", "acc"), ("recv", "tx", "acc"), ("at", "tx", "mer"),
        ("rev_sent", "acc", "tx"), ("rev_recv", "acc", "tx"), ("rev_at", "mer", "tx")]

BROWS = 2048
BSHIFT = 11
ACC_ROWS = 2064
SLAB = 2056
NB = {"tx": 49, "acc": 25, "mer": 5}
NBR = [NB[d] for (_, _, d) in RELS]
NBMAX = 49
NL = 64

NW = 32
TCH = 3200
EP = TCH * 32
KCH = 128
NRE = TCH // KCH
NRI = NRE + NBMAX
RSZ = TCH + NBMAX * KCH
GTRASH = 6 * NW * RSZ
LSZ = GTRASH + TCH + 256
PADV = 1 << 22


@functools.cache
def _mesh():
    return plsc.VectorSubcoreMesh(core_axis_name="c", subcore_axis_name="s",
                                  num_cores=2, num_subcores=16)


def _meta_body(e_ref, nb_ref, tri_ref, offs_ref, cnts_ref):
    r = pl.program_id(0)
    t = pl.program_id(1)
    d = e_ref[0, 0, 0]
    bkt = lax.shift_right_logical(d, BSHIFT)
    io = lax.broadcasted_iota(jnp.int32, (NL, TCH), 0)
    oh = (bkt == io).astype(jnp.float32)
    cnt = jnp.sum(oh, axis=1)[None, :]
    cnti = cnt.astype(jnp.int32)
    cnt_pad = ((cnti + (KCH - 1)) // KCH) * KCH
    pref = (cnt_pad.astype(jnp.float32) @ tri_ref[...]).astype(jnp.int32)
    offs = (r * NW + t) * RSZ + pref
    nbv = nb_ref[0]
    lanes = lax.broadcasted_iota(jnp.int32, (1, NL), 1)
    valid = lanes < nbv
    offs_ref[0, 0] = jnp.where(valid, offs, jnp.int32(GTRASH))
    cnts_ref[0, 0] = jnp.where(valid, cnt_pad, 0)


def _meta(e5, nb_arr, tri):
    return pl.pallas_call(
        _meta_body,
        grid=(6, NW),
        in_specs=[pl.BlockSpec((1, 1, 1, 1, TCH), lambda r, t: (r, 1, t, 0, 0)),
                  pl.BlockSpec((1, 1, NL), lambda r, t: (r, 0, 0)),
                  pl.BlockSpec((NL, NL), lambda r, t: (0, 0))],
        out_specs=[pl.BlockSpec((1, 1, 1, NL), lambda r, t: (r, t, 0, 0)),
                   pl.BlockSpec((1, 1, 1, NL), lambda r, t: (r, t, 0, 0))],
        out_shape=[jax.ShapeDtypeStruct((6, NW, 1, NL), jnp.int32),
                   jax.ShapeDtypeStruct((6, NW, 1, NL), jnp.int32)],
    )(e5, nb_arr, tri)


def _prep_body(e_ref, offs_ref, cnts_ref, srcs_ref, dsts_ref,
               sv, dv, offv, cntv, idx2, pos_s):
    cid = lax.axis_index("c")
    sid = lax.axis_index("s")
    t = cid * 16 + sid
    base = t * TCH
    iota16 = lax.iota(jnp.int32, 16)
    zeros16 = jnp.zeros((16,), jnp.int32)
    for r in range(6):
        nb = NBR[r]
        pltpu.sync_copy(e_ref.at[pl.ds(2 * r * EP + base, TCH)],
                        sv.at[pl.ds(0, TCH)])
        pltpu.sync_copy(e_ref.at[pl.ds((2 * r + 1) * EP + base, TCH)],
                        dv.at[pl.ds(0, TCH)])
        pltpu.sync_copy(offs_ref.at[pl.ds((r * NW + t) * NL, NL)], offv)
        pltpu.sync_copy(cnts_ref.at[pl.ds((r * NW + t) * NL, NL)], cntv)
        ov = [offv[pl.ds(16 * i, 16)] for i in range(NL // 16)]
        cv = [cntv[pl.ds(16 * i, 16)] for i in range(NL // 16)]
        for lane in range(NL - 1):
            pos_s[lane] = ov[lane // 16][lane % 16]
        pos_s[NL - 1] = jnp.int32(GTRASH)

        def group(g, _):
            dvv = dv[pl.ds(g * 16, 16)]
            bkt = jnp.minimum(lax.shift_right_logical(dvv, BSHIFT), NL - 1)
            idxv = zeros16
            for lane in range(16):
                b_l = bkt[lane]
                p_l = pos_s[b_l]
                pos_s[b_l] = p_l + 1
                idxv = jnp.where(iota16 == lane, p_l, idxv)
            idx2[g // 8, pl.ds((g % 8) * 16, 16)] = idxv
            return 0

        lax.fori_loop(0, TCH // 16, group, 0)
        for b in range(nb):
            p0 = pos_s[b]
            endb = ov[b // 16][b % 16] + cv[b // 16][b % 16]
            sdst = jnp.full((16,), b * BROWS + BROWS, jnp.int32)

            def padq(q, _, _b=b, _p0=p0, _endb=endb, _sdst=sdst):
                slot = _p0 + q * 16 + iota16
                idx2[NRE + _b, pl.ds(q * 16, 16)] = jnp.where(
                    slot < _endb, slot, jnp.int32(GTRASH))
                sv[pl.ds(TCH + _b * KCH + q * 16, 16)] = zeros16
                dv[pl.ds(TCH + _b * KCH + q * 16, 16)] = _sdst
                return 0

            lax.fori_loop(0, KCH // 16, padq, 0)

        def scatter_out(sem):
            def fire(j, _):
                pltpu.async_copy(sv.at[pl.ds(j * KCH, KCH)],
                                 srcs_ref.at[idx2.at[j]], sem)
                pltpu.async_copy(dv.at[pl.ds(j * KCH, KCH)],
                                 dsts_ref.at[idx2.at[j]], sem)
                return 0

            lax.fori_loop(0, NRE + nb, fire, 0)

            def drain(j, _):
                pltpu.make_async_copy(srcs_ref.at[pl.ds(0, KCH)],
                                      sv.at[pl.ds(0, KCH)], sem).wait()
                pltpu.make_async_copy(dsts_ref.at[pl.ds(0, KCH)],
                                      dv.at[pl.ds(0, KCH)], sem).wait()
                return 0

            lax.fori_loop(0, NRE + nb, drain, 0)

        pl.run_scoped(scatter_out, pltpu.SemaphoreType.DMA)


@functools.cache
def _make_prep():
    return functools.partial(
        pl.kernel,
        out_type=[jax.ShapeDtypeStruct((LSZ,), jnp.int32),
                  jax.ShapeDtypeStruct((LSZ,), jnp.int32)],
        mesh=_mesh(),
        scratch_types=[pltpu.VMEM((RSZ,), jnp.int32),
                       pltpu.VMEM((RSZ,), jnp.int32),
                       pltpu.VMEM((NL,), jnp.int32),
                       pltpu.VMEM((NL,), jnp.int32),
                       pltpu.VMEM((NRI, KCH), jnp.int32),
                       pltpu.SMEM((NL,), jnp.int32)],
        name="gat_prep",
    )(_prep_body)


def _agg_body(nb, roff, xs_ref, ad_ref, srcs_ref, dsts_ref, offs_ref, cnts_ref,
              num_ref, den_ref, offs_v, cnts_v, src_idx, dst_idx, exb, xsg2,
              adg, zb, zbd, acc_n, acc_d, offs_s, cnts_s):
    cid = lax.axis_index("c")
    sid = lax.axis_index("s")
    iotam = lax.iota(jnp.int32, 16)
    lmask = [jnp.where(iotam == h, jnp.float32(1.0), jnp.float32(0.0))
             for h in range(H)]
    pltpu.sync_copy(offs_ref.at[pl.ds(roff * NW * NL, NW * NL)], offs_v)
    pltpu.sync_copy(cnts_ref.at[pl.ds(roff * NW * NL, NW * NL)], cnts_v)
    for th in range(2):
        row0 = pl.multiple_of((sid + 16 * th) * NL, 16)
        orow = [offs_v[pl.ds(row0 + 16 * i, 16)] for i in range(NL // 16)]
        crow = [cnts_v[pl.ds(row0 + 16 * i, 16)] for i in range(NL // 16)]
        for lane in range(NL):
            offs_s[th * NL + lane] = orow[lane // 16][lane % 16]
            cnts_s[th * NL + lane] = crow[lane // 16][lane % 16]
    zrow = jnp.zeros((16,), jnp.float32)

    def zrow_body(i, _):
        for j in range(8):
            zb[i, pl.ds(j * 16, 16)] = zrow
        zbd[i, :] = zrow
        return 0

    lax.fori_loop(0, 16, zrow_body, 0)

    nb_mine = (nb + 1 - cid) // 2

    def bucket_body(bi, _):
        b = cid + 2 * bi
        base_rows = b * BROWS

        def zcp(q, _z):
            row = sid * (BROWS // 16) + q * 16
            pltpu.sync_copy(zb, acc_n.at[pl.ds(row, 16)])
            pltpu.sync_copy(zbd, acc_d.at[pl.ds(row, 16)])
            return 0

        lax.fori_loop(0, BROWS // 256, zcp, 0)

        @pl.when(sid == 0)
        def _():
            pltpu.sync_copy(zb, acc_n.at[pl.ds(BROWS, ACC_ROWS - BROWS)])
            pltpu.sync_copy(zbd, acc_d.at[pl.ds(BROWS, ACC_ROWS - BROWS)])

        plsc.subcore_barrier()
        for th in range(2):
            off_t = offs_s[th * NL + b]
            cnt_t = cnts_s[th * NL + b]
            nch = cnt_t // KCH

            def chunk(j, _c):
                sbase = pl.multiple_of(off_t + j * KCH, 8)
                pltpu.sync_copy(srcs_ref.at[pl.ds(sbase, KCH)], src_idx)
                pltpu.sync_copy(dsts_ref.at[pl.ds(sbase, KCH)], dst_idx.at[0])
                pltpu.sync_copy(xs_ref.at[src_idx], xsg2)
                pltpu.sync_copy(ad_ref.at[dst_idx.at[0]], adg)
                for g in range(KCH // 16):
                    dst_idx[0, pl.ds(g * 16, 16)] = (
                        dst_idx[0, pl.ds(g * 16, 16)] - base_rows)

                def edge(k, _e):
                    evc = jnp.zeros((16,), jnp.float32)
                    for jj in range(8):
                        av = (xsg2[k, pl.ds(HC + jj * 16, 16)] +
                              adg[k, pl.ds(jj * 16, 16)])
                        av = jnp.where(av > 0, av, av * jnp.float32(0.2))
                        ev = jnp.exp(av)
                        adg[k, pl.ds(jj * 16, 16)] = (
                            xsg2[k, pl.ds(jj * 16, 16)] * ev)
                        if jj % 2 == 0:
                            evc = evc + ev * lmask[jj // 2]
                    exb[k, :] = evc
                    return 0

                lax.fori_loop(0, KCH, edge, 0)
                pltpu.sync_copy(adg, acc_n.at[dst_idx.at[0]], add=True)
                pltpu.sync_copy(exb, acc_d.at[dst_idx.at[0]], add=True)
                return 0

            lax.fori_loop(0, nch, chunk, 0)
        plsc.subcore_barrier()
        pltpu.sync_copy(acc_n.at[pl.ds(sid * (BROWS // 16), BROWS // 16)],
                        num_ref.at[pl.ds(base_rows + sid * (BROWS // 16),
                                         BROWS // 16)])
        pltpu.sync_copy(acc_d.at[pl.ds(sid * (BROWS // 16), BROWS // 16)],
                        den_ref.at[pl.ds(base_rows + sid * (BROWS // 16),
                                         BROWS // 16)])
        plsc.subcore_barrier()
        return 0

    lax.fori_loop(0, nb_mine, bucket_body, 0)


def _make_agg(nb, roff):
    @functools.partial(
        pl.kernel,
        out_type=[jax.ShapeDtypeStruct((nb * BROWS, HC), jnp.float32),
                  jax.ShapeDtypeStruct((nb * BROWS, 16), jnp.float32)],
        mesh=_mesh(),
        scratch_types=[pltpu.VMEM((NW * NL,), jnp.int32),
                       pltpu.VMEM((NW * NL,), jnp.int32),
                       pltpu.VMEM((KCH,), jnp.int32),
                       pltpu.VMEM((1, KCH), jnp.int32),
                       pltpu.VMEM((KCH, 16), jnp.float32),
                       pltpu.VMEM((KCH, 2 * HC), jnp.float32),
                       pltpu.VMEM((KCH, HC), jnp.float32),
                       pltpu.VMEM((16, HC), jnp.float32),
                       pltpu.VMEM((16, 16), jnp.float32),
                       pltpu.VMEM_SHARED((ACC_ROWS, HC), jnp.float32),
                       pltpu.VMEM_SHARED((ACC_ROWS, 16), jnp.float32),
                       pltpu.SMEM((2 * NL,), jnp.int32),
                       pltpu.SMEM((2 * NL,), jnp.int32)],
        name="gat_agg",
    )
    def k(xs_ref, ad_ref, srcs_ref, dsts_ref, offs_ref, cnts_ref,
          num_ref, den_ref, offs_v, cnts_v, src_idx, dst_idx, exb, xsg2,
          adg, zb, zbd, acc_n, acc_d, offs_s, cnts_s):
        _agg_body(nb, roff, xs_ref, ad_ref, srcs_ref, dsts_ref, offs_ref,
                  cnts_ref, num_ref, den_ref, offs_v, cnts_v, src_idx,
                  dst_idx, exb, xsg2, adg, zb, zbd, acc_n, acc_d,
                  offs_s, cnts_s)

    return k


BLK = 512


def _grid(n):
    return (n + BLK - 1) // BLK


def _proj_body(x_ref, w_ref, b_ref, o_ref):
    o_ref[...] = jnp.maximum(x_ref[...] @ w_ref[...] + b_ref[...], 0.0)


def _proj(x, w, b):
    n = x.shape[0]
    return pl.pallas_call(
        _proj_body,
        grid=(_grid(n),),
        in_specs=[pl.BlockSpec((BLK, x.shape[1]), lambda i: (i, 0)),
                  pl.BlockSpec(w.shape, lambda i: (0, 0)),
                  pl.BlockSpec((1, w.shape[1]), lambda i: (0, 0))],
        out_specs=pl.BlockSpec((BLK, w.shape[1]), lambda i: (i, 0)),
        out_shape=jax.ShapeDtypeStruct((n, w.shape[1]), jnp.float32),
    )(x, w, b.reshape(1, -1))


def _make_pre_src_body(nmat):
    def body(*refs):
        x = refs[0][...]
        for i in range(nmat):
            xw = x @ refs[1 + 2 * i][...]
            aw = x @ refs[2 + 2 * i][...]
            refs[1 + 2 * nmat + i][...] = jnp.concatenate([xw, aw], axis=1)
    return body


def _pre_src(h, wpairs):
    n, inch = h.shape
    nmat = len(wpairs)
    ws = [w for pair in wpairs for w in pair]
    return pl.pallas_call(
        _make_pre_src_body(nmat),
        grid=(_grid(n),),
        in_specs=[pl.BlockSpec((BLK, inch), lambda i: (i, 0))] +
                 [pl.BlockSpec(w.shape, lambda i: (0, 0)) for w in ws],
        out_specs=[pl.BlockSpec((BLK, 2 * HC), lambda i: (i, 0))
                   for _ in range(nmat)],
        out_shape=[jax.ShapeDtypeStruct((n, 2 * HC), jnp.float32)
                   for _ in range(nmat)],
    )(h, *ws)


def _make_pre_dst_body(nmat):
    def body(*refs):
        x = refs[0][...]
        for i in range(nmat):
            refs[1 + nmat + i][...] = x @ refs[1 + i][...]
    return body


def _pre_dst(h, was, n_pad):
    n, inch = h.shape
    nmat = len(was)
    nlast = (n + BLK - 1) // BLK - 1
    return pl.pallas_call(
        _make_pre_dst_body(nmat),
        grid=(_grid(n_pad),),
        in_specs=[pl.BlockSpec((BLK, inch),
                               lambda i: (jnp.minimum(i, nlast), 0))] +
                 [pl.BlockSpec(w.shape, lambda i: (0, 0)) for w in was],
        out_specs=[pl.BlockSpec((BLK, HC), lambda i: (i, 0))
                   for _ in range(nmat)],
        out_shape=[jax.ShapeDtypeStruct((n_pad, HC), jnp.float32)
                   for _ in range(nmat)],
    )(h, *was)


def _make_post_body(nrel, residual):
    def body(*refs):
        acc = None
        for i in range(nrel):
            num = refs[2 * i][...]
            den = refs[2 * i + 1][...][:, :H]
            dexp = jnp.broadcast_to(den[:, :, None], den.shape + (C,))
            dexp = dexp.reshape(den.shape[0], HC)
            term = num / (dexp + 1e-16)
            acc = term if acc is None else acc + term
        mult = refs[2 * nrel][...]
        add = refs[2 * nrel + 1][...]
        y = jnp.maximum(acc * mult + add, 0.0)
        if residual:
            y = y + refs[2 * nrel + 2][...]
        refs[-1][...] = y
    return body


def _post(nums, dens, mult, add, res, n):
    nrel = len(nums)
    ops = []
    specs = []
    for num, den in zip(nums, dens):
        ops += [num, den]
        specs += [pl.BlockSpec((BLK, HC), lambda i: (i, 0)),
                  pl.BlockSpec((BLK, 16), lambda i: (i, 0))]
    ops += [mult.reshape(1, HC), add.reshape(1, HC)]
    specs += [pl.BlockSpec((1, HC), lambda i: (0, 0)),
              pl.BlockSpec((1, HC), lambda i: (0, 0))]
    if res is not None:
        ops.append(res)
        specs.append(pl.BlockSpec((BLK, HC), lambda i: (i, 0)))
    return pl.pallas_call(
        _make_post_body(nrel, res is not None),
        grid=(_grid(n),),
        in_specs=specs,
        out_specs=pl.BlockSpec((BLK, HC), lambda i: (i, 0)),
        out_shape=jax.ShapeDtypeStruct((n, HC), jnp.float32),
    )(*ops)


def _head_body(x_ref, w1_ref, b1_ref, w2_ref, b2_ref, o_ref):
    z = jnp.maximum(x_ref[...] @ w1_ref[...] + b1_ref[...], 0.0)
    s = jnp.sum(z * w2_ref[...], axis=1, keepdims=True) + b2_ref[...]
    o_ref[...] = jnp.broadcast_to(s, (s.shape[0], 8))


def _head(h, p1, p2):
    n = h.shape[0]
    out = pl.pallas_call(
        _head_body,
        grid=(_grid(n),),
        in_specs=[pl.BlockSpec((BLK, HC), lambda i: (i, 0)),
                  pl.BlockSpec((HC, 32), lambda i: (0, 0)),
                  pl.BlockSpec((1, 32), lambda i: (0, 0)),
                  pl.BlockSpec((1, 32), lambda i: (0, 0)),
                  pl.BlockSpec((1, 1), lambda i: (0, 0))],
        out_specs=pl.BlockSpec((BLK, 8), lambda i: (i, 0)),
        out_shape=jax.ShapeDtypeStruct((n, 8), jnp.float32),
    )(h, p1["W"], p1["b"].reshape(1, 32), p2["W"].reshape(1, 32),
      p2["b"].reshape(1, 1))
    return out[:, 0]


def _att_w(p, key):
    w3 = p["W"].reshape(-1, H, C)
    wa = (w3 * p[key][None, :, :]).sum(-1)
    return jnp.repeat(wa, C, axis=1)


def kernel(x_tx, x_acc, x_mer, e_sent, e_recv, e_at,
           e_rev_sent, e_rev_recv, e_rev_at, params):
    edges = [e_sent, e_recv, e_at, e_rev_sent, e_rev_recv, e_rev_at]
    e_all = jnp.stack(edges).astype(jnp.int32)
    e_all = jnp.pad(e_all, ((0, 0), (0, 0), (0, EP - E)),
                    constant_values=PADV)
    nb_arr = jnp.broadcast_to(
        jnp.array(NBR, jnp.int32)[:, None, None], (6, 1, NL))
    tri = jnp.triu(jnp.ones((NL, NL), jnp.float32), 1)
    offs5, cnts5 = _meta(e_all.reshape(6, 2, NW, 1, TCH), nb_arr, tri)
    offs = offs5.reshape(-1)
    cnts = cnts5.reshape(-1)
    srcs, dsts = _make_prep()(e_all.reshape(-1), offs, cnts)

    h = {"tx": _proj(x_tx, params["proj_tx"]["W"], params["proj_tx"]["b"]),
         "acc": _proj(x_acc, params["proj_acc"]["W"], params["proj_acc"]["b"]),
         "mer": _proj(x_mer, params["proj_mer"]["W"], params["proj_mer"]["b"])}

    aggs = {}
    for r, (name, s, d) in enumerate(RELS):
        aggs[name] = _make_agg(NB[d], r)

    for l in range(2):
        per_rel = {name: params["conv%d_%s" % (l, name)]
                   for (name, s, d) in RELS}
        plans = {}
        for typ in ("tx", "acc", "mer"):
            spairs, stags = [], []
            for name, s, d in RELS:
                if s == typ:
                    spairs.append((per_rel[name]["W"],
                                   _att_w(per_rel[name], "att_src")))
                    stags.append(name)
            outs = _pre_src(h[typ], spairs)
            for name, o in zip(stags, outs):
                plans[("xsa", name)] = o
            was, dtags = [], []
            for name, s, d in RELS:
                if d == typ:
                    was.append(_att_w(per_rel[name], "att_dst"))
                    dtags.append(name)
            n_pad = NB[typ] * BROWS + 256
            outs = _pre_dst(h[typ], was, n_pad)
            for name, o in zip(dtags, outs):
                plans[("ad", name)] = o

        nums, dens = {}, {}
        for r, (name, s, d) in enumerate(RELS):
            num, den = aggs[name](
                plans[("xsa", name)], plans[("ad", name)],
                srcs, dsts, offs, cnts)
            nums[name] = num
            dens[name] = den

        out = {}
        for typ in ("tx", "acc", "mer"):
            rel_n = [name for (name, s, d) in RELS if d == typ]
            b_sum = sum(per_rel[name]["b"] for name in rel_n)
            if typ == "tx":
                bn = params["bn%d" % l]
                scale = bn["gamma"] / jnp.sqrt(bn["var"] + 1e-5)
                mult = scale
                add = b_sum * scale + bn["beta"] - bn["mean"] * scale
            else:
                mult = jnp.ones((HC,), jnp.float32)
                add = b_sum
            res = h[typ] if l > 0 else None
            out[typ] = _post([nums[n_][:NNODE[typ]] for n_ in rel_n],
                             [dens[n_][:NNODE[typ]] for n_ in rel_n],
                             mult, add, res, NNODE[typ])
        h = out

    return _head(h["tx"], params["head1"], params["head2"])
